# baseline (speedup 1.0000x reference)
import functools

import jax
import jax.numpy as jnp
from jax import lax
from jax.experimental import pallas as pl
from jax.experimental.pallas import tpu as pltpu
from jax.experimental.pallas import tpu_sc as plsc

D = 128
LANES = 16
NW = 32


SB = 16


def _gsum_body(*refs, n_chunks, pw, n_sb, chunk, has_src):
    if has_src:
        (table, idxb, src, out, idx_v, rows_v, src_v,
         semg0, semg1, semi, semo0, semo1) = refs
    else:
        (table, idxb, out, idx_v, rows_v,
         semg0, semg1, semi, semo0, semo1) = refs
        src = src_v = None
    cid = lax.axis_index("c")
    sid = lax.axis_index("s")
    wid = sid * 2 + cid
    w0 = wid * pw
    semg = (semg0, semg1)
    semo = (semo0, semo1)
    nv = D // LANES

    def idx_desc(q):
        return pltpu.make_async_copy(
            idxb.at[pl.ds(w0 + q * SB, SB)], idx_v.at[q % 2], semi)

    def gather_descs(t, bb):
        q = t // SB
        u = t % SB
        ds = [pltpu.make_async_copy(table.at[idx_v.at[q % 2, u, j]],
                                    rows_v.at[bb, j], semg[bb])
              for j in range(4)]
        if has_src:
            ds.append(pltpu.make_async_copy(
                src.at[pl.ds((w0 + t) * chunk, chunk)], src_v.at[bb],
                semg[bb]))
        return ds

    def out_desc(t, bb):
        return pltpu.make_async_copy(
            rows_v.at[bb, 3], out.at[pl.ds((w0 + t) * chunk, chunk)],
            semo[bb])

    def fire_gathers(t, bb):
        @pl.when(jnp.logical_and(t >= 2, w0 + t - 2 < n_chunks))
        def _():
            out_desc(t - 2, bb).wait()

        for d in gather_descs(t, bb):
            d.start()

    def process(t, b):
        for d in gather_descs(t, b):
            d.wait()

        def row_body(r, rc):
            for k in range(nv):
                s = pl.ds(k * LANES, LANES)
                vs = [rows_v[b, j, r, s] for j in range(4)]
                v = (vs[0] + vs[1]) + (vs[2] + vs[3])
                if has_src:
                    v = jnp.maximum(v + src_v[b, r, s], 0)
                rows_v[b, 3, r, s] = v
            return rc

        lax.fori_loop(0, chunk, row_body, 0)

        @pl.when(w0 + t < n_chunks)
        def _():
            out_desc(t, b).start()

    pltpu.sync_copy(idxb.at[pl.ds(w0, SB)], idx_v.at[0])
    if n_sb > 1:
        idx_desc(1).start()
    fire_gathers(0, 0)

    def pair_body(u2, carry):
        for b in (0, 1):
            t = u2 * 2 + b

            @pl.when(t < pw)
            def _():
                tn = t + 1

                @pl.when(tn < pw)
                def _():
                    if n_sb > 1:
                        @pl.when(tn % SB == 0)
                        def _():
                            q = tn // SB
                            idx_desc(q).wait()

                            @pl.when(q + 1 < n_sb)
                            def _():
                                idx_desc(q + 1).start()

                    fire_gathers(tn, 1 - b)

                process(t, b)

        return carry

    lax.fori_loop(0, (pw + 1) // 2, pair_body, 0)

    for tl in (pw - 2, pw - 1):
        if tl >= 0:
            @pl.when(w0 + tl < n_chunks)
            def _():
                out_desc(tl, tl % 2).wait()


def _gsum_sc(table, idxb_pad, n_out, chunk, src=None):
    n_chunks = n_out // chunk
    pw = -(-n_chunks // NW)
    n_sb = -(-pw // SB)
    has_src = src is not None
    mesh = plsc.VectorSubcoreMesh(core_axis_name="c", subcore_axis_name="s",
                                  num_cores=2, num_subcores=16)
    scratch = [
        pltpu.VMEM((2, SB, 4, chunk), jnp.int32),
        pltpu.VMEM((2, 4, chunk, D), jnp.float32),
    ]
    if has_src:
        scratch.append(pltpu.VMEM((2, chunk, D), jnp.float32))
    scratch += [pltpu.SemaphoreType.DMA] * 5
    kern = pl.kernel(
        functools.partial(_gsum_body, n_chunks=n_chunks, pw=pw, n_sb=n_sb,
                          chunk=chunk, has_src=has_src),
        out_type=jax.ShapeDtypeStruct((n_out, D), jnp.float32),
        mesh=mesh,
        scratch_types=scratch,
        name=("gsum_add_relu" if has_src else "gsum"),
    )
    return (kern(table, idxb_pad, src) if has_src
            else kern(table, idxb_pad))


def _pad_chunks(idxb, chunk):
    n_chunks = idxb.shape[0]
    pw = -(-n_chunks // NW)
    n_sb = -(-pw // SB)
    total = NW * n_sb * SB
    if total > n_chunks:
        idxb = jnp.concatenate(
            [idxb, jnp.zeros((total - n_chunks,) + idxb.shape[1:],
                             jnp.int32)], axis=0)
    return idxb


def _mm2_body(x_ref, wi_ref, wh_ref, inp_ref, z0_ref):
    inp = lax.dot_general(
        x_ref[...], wi_ref[...], (((1,), (1,)), ((), ())),
        preferred_element_type=jnp.float32)
    inp_ref[...] = inp
    z0_ref[...] = lax.dot_general(
        jnp.maximum(inp, 0.0), wh_ref[...], (((1,), (1,)), ((), ())),
        preferred_element_type=jnp.float32)


def _mm2(x, wi, wh, bm):
    n, k = x.shape
    return pl.pallas_call(
        _mm2_body,
        grid=(n // bm,),
        in_specs=[pl.BlockSpec((bm, k), lambda i: (i, 0)),
                  pl.BlockSpec((D, k), lambda i: (0, 0)),
                  pl.BlockSpec((D, D), lambda i: (0, 0))],
        out_specs=[pl.BlockSpec((bm, D), lambda i: (i, 0)),
                   pl.BlockSpec((bm, D), lambda i: (i, 0))],
        out_shape=[jax.ShapeDtypeStruct((n, D), jnp.float32),
                   jax.ShapeDtypeStruct((n, D), jnp.float32)],
    )(x, wi, wh)


def _out_body(af_ref, a_ref, wa_ref, wb_ref, b_ref, o_ref, *, bm, per):
    h = lax.dot_general(af_ref[...], wa_ref[...], (((1,), (1,)), ((), ())),
                        preferred_element_type=jnp.float32)
    h = h + lax.dot_general(a_ref[...], wb_ref[...], (((1,), (1,)), ((), ())),
                            preferred_element_type=jnp.float32)
    h = jnp.maximum(h + b_ref[...], 0.0)
    o_ref[...] = h.reshape(bm // per, per, D).sum(axis=1)


def _out_stage(af, a, wa, wb, b, bm, per):
    n = af.shape[0]
    ka = af.shape[1]
    n_mol_blk = bm // per
    return pl.pallas_call(
        functools.partial(_out_body, bm=bm, per=per),
        grid=(n // bm,),
        in_specs=[pl.BlockSpec((bm, ka), lambda i: (i, 0)),
                  pl.BlockSpec((bm, D), lambda i: (i, 0)),
                  pl.BlockSpec((D, ka), lambda i: (0, 0)),
                  pl.BlockSpec((D, D), lambda i: (0, 0)),
                  pl.BlockSpec((1, D), lambda i: (0, 0))],
        out_specs=pl.BlockSpec((n_mol_blk, D), lambda i: (i, 0)),
        out_shape=jax.ShapeDtypeStruct((n // per, D), jnp.float32),
    )(af, a, wa, wb, b)


def _pick(n, pref):
    return pref if n % pref == 0 else n


def kernel(atom_features, f_ini_atoms_bonds, atom_to_incoming_bonds, mapping,
           global_features, molecules_unbatch_key, W_i, W_h, W_o_w, W_o_b):
    n_bonds = f_ini_atoms_bonds.shape[0]
    n_atoms = atom_features.shape[0]
    afdim = atom_features.shape[1]

    cb = _pick(n_bonds, 80)
    ca = _pick(n_atoms, 80)
    bmb = _pick(n_bonds, 2000)
    bma = _pick(n_atoms, 2000)

    idxb_bonds = _pad_chunks(mapping.astype(jnp.int32).reshape(
        n_bonds // cb, cb, 4).transpose(0, 2, 1), cb)
    idxb_atoms = _pad_chunks(atom_to_incoming_bonds.astype(jnp.int32).reshape(
        n_atoms // ca, ca, 4).transpose(0, 2, 1), ca)

    inp, z0 = _mm2(f_ini_atoms_bonds, W_i, W_h, bm=bmb)
    zm = _gsum_sc(z0, idxb_bonds, n_bonds, cb)
    h = _gsum_sc(zm, idxb_bonds, n_bonds, cb, src=inp)
    a = _gsum_sc(h, idxb_atoms, n_atoms, ca)

    wa = W_o_w[:, :afdim]
    wb = W_o_w[:, afdim:]
    mol_sum = _out_stage(atom_features, a, wa, wb,
                         W_o_b.reshape(1, D), bm=bma, per=25)
    mol = mol_sum / molecules_unbatch_key
    return jnp.concatenate([mol, global_features], axis=1)


# speedup vs baseline: 1.0059x; 1.0059x over previous
import functools

import jax
import jax.numpy as jnp
from jax import lax
from jax.experimental import pallas as pl
from jax.experimental.pallas import tpu as pltpu
from jax.experimental.pallas import tpu_sc as plsc

D = 128
LANES = 16
NW = 32


def _gsum_body(*refs, n_chunks, pw, n_sb, sb, chunk, has_src):
    if has_src:
        (table, idxb, src, out, idx_v, rows_v, src_v, acc_v,
         semg0, semg1, semi, semo0, semo1) = refs
    else:
        (table, idxb, out, idx_v, rows_v, acc_v,
         semg0, semg1, semi, semo0, semo1) = refs
        src = src_v = None
    cid = lax.axis_index("c")
    sid = lax.axis_index("s")
    wid = sid * 2 + cid
    w0 = wid * pw
    semg = (semg0, semg1)
    semo = (semo0, semo1)
    n_acc = 1 if has_src else 2
    nv = D // LANES

    def idx_desc(q):
        return pltpu.make_async_copy(
            idxb.at[pl.ds(w0 + q * sb, sb)], idx_v.at[q % 2], semi)

    def gather_descs(t, bb):
        q = t // sb
        u = t % sb
        ds = [pltpu.make_async_copy(table.at[idx_v.at[q % 2, u, j]],
                                    rows_v.at[bb, j], semg[bb])
              for j in range(4)]
        if has_src:
            ds.append(pltpu.make_async_copy(
                src.at[pl.ds((w0 + t) * chunk, chunk)], src_v.at[bb],
                semg[bb]))
        return ds

    def out_desc(t, a):
        return pltpu.make_async_copy(
            acc_v.at[a], out.at[pl.ds((w0 + t) * chunk, chunk)], semo[a])

    def fire_gathers(t, bb):
        for d in gather_descs(t, bb):
            d.start()

    def process(t, b):
        a = 0 if has_src else b
        for d in gather_descs(t, b):
            d.wait()

        @pl.when(jnp.logical_and(t >= n_acc, w0 + t - n_acc < n_chunks))
        def _():
            out_desc(t - n_acc, a).wait()

        def row_body(r, rc):
            for k in range(nv):
                s = pl.ds(k * LANES, LANES)
                vs = [rows_v[b, j, r, s] for j in range(4)]
                v = (vs[0] + vs[1]) + (vs[2] + vs[3])
                if has_src:
                    v = jnp.maximum(v + src_v[b, r, s], 0)
                acc_v[a, r, s] = v
            return rc

        lax.fori_loop(0, chunk, row_body, 0)

        @pl.when(w0 + t < n_chunks)
        def _():
            out_desc(t, a).start()

    pltpu.sync_copy(idxb.at[pl.ds(w0, sb)], idx_v.at[0])
    if n_sb > 1:
        idx_desc(1).start()
    fire_gathers(0, 0)

    def pair_body(u2, carry):
        for b in (0, 1):
            t = u2 * 2 + b

            @pl.when(t < pw)
            def _():
                tn = t + 1

                @pl.when(tn < pw)
                def _():
                    if n_sb > 1:
                        @pl.when(tn % sb == 0)
                        def _():
                            q = tn // sb
                            idx_desc(q).wait()

                            @pl.when(q + 1 < n_sb)
                            def _():
                                idx_desc(q + 1).start()

                    fire_gathers(tn, 1 - b)

                process(t, b)

        return carry

    lax.fori_loop(0, (pw + 1) // 2, pair_body, 0)

    for i in range(n_acc):
        tl = pw - 1 - i
        if tl >= 0:
            a = 0 if has_src else tl % 2

            @pl.when(w0 + tl < n_chunks)
            def _():
                out_desc(tl, a).wait()


def _gsum_sc(table, idxb_pad, n_out, chunk, src=None):
    n_chunks = n_out // chunk
    pw = -(-n_chunks // NW)
    has_src = src is not None
    sb = 8 if has_src else 16
    n_sb = -(-pw // sb)
    mesh = plsc.VectorSubcoreMesh(core_axis_name="c", subcore_axis_name="s",
                                  num_cores=2, num_subcores=16)
    scratch = [
        pltpu.VMEM((2, sb, 4, chunk), jnp.int32),
        pltpu.VMEM((2, 4, chunk, D), jnp.float32),
    ]
    if has_src:
        scratch.append(pltpu.VMEM((2, chunk, D), jnp.float32))
    scratch.append(pltpu.VMEM((1 if has_src else 2, chunk, D), jnp.float32))
    scratch += [pltpu.SemaphoreType.DMA] * 5
    kern = pl.kernel(
        functools.partial(_gsum_body, n_chunks=n_chunks, pw=pw, n_sb=n_sb,
                          sb=sb, chunk=chunk, has_src=has_src),
        out_type=jax.ShapeDtypeStruct((n_out, D), jnp.float32),
        mesh=mesh,
        scratch_types=scratch,
        name=("gsum_add_relu" if has_src else "gsum"),
    )
    return (kern(table, idxb_pad, src) if has_src
            else kern(table, idxb_pad))


def _pad_chunks(idxb):
    n_chunks = idxb.shape[0]
    pw = -(-n_chunks // NW)
    total = NW * (-(-pw // 16)) * 16
    if total > n_chunks:
        idxb = jnp.concatenate(
            [idxb, jnp.zeros((total - n_chunks,) + idxb.shape[1:],
                             jnp.int32)], axis=0)
    return idxb


def _mm2_body(x_ref, wi_ref, wh_ref, inp_ref, z0_ref):
    inp = lax.dot_general(
        x_ref[...], wi_ref[...], (((1,), (1,)), ((), ())),
        preferred_element_type=jnp.float32)
    inp_ref[...] = inp
    z0_ref[...] = lax.dot_general(
        jnp.maximum(inp, 0.0), wh_ref[...], (((1,), (1,)), ((), ())),
        preferred_element_type=jnp.float32)


def _mm2(x, wi, wh, bm):
    n, k = x.shape
    return pl.pallas_call(
        _mm2_body,
        grid=(n // bm,),
        in_specs=[pl.BlockSpec((bm, k), lambda i: (i, 0)),
                  pl.BlockSpec((D, k), lambda i: (0, 0)),
                  pl.BlockSpec((D, D), lambda i: (0, 0))],
        out_specs=[pl.BlockSpec((bm, D), lambda i: (i, 0)),
                   pl.BlockSpec((bm, D), lambda i: (i, 0))],
        out_shape=[jax.ShapeDtypeStruct((n, D), jnp.float32),
                   jax.ShapeDtypeStruct((n, D), jnp.float32)],
    )(x, wi, wh)


def _out_body(af_ref, a_ref, wa_ref, wb_ref, b_ref, o_ref, *, bm, per):
    h = lax.dot_general(af_ref[...], wa_ref[...], (((1,), (1,)), ((), ())),
                        preferred_element_type=jnp.float32)
    h = h + lax.dot_general(a_ref[...], wb_ref[...], (((1,), (1,)), ((), ())),
                            preferred_element_type=jnp.float32)
    h = jnp.maximum(h + b_ref[...], 0.0)
    o_ref[...] = h.reshape(bm // per, per, D).sum(axis=1)


def _out_stage(af, a, wa, wb, b, bm, per):
    n = af.shape[0]
    ka = af.shape[1]
    n_mol_blk = bm // per
    return pl.pallas_call(
        functools.partial(_out_body, bm=bm, per=per),
        grid=(n // bm,),
        in_specs=[pl.BlockSpec((bm, ka), lambda i: (i, 0)),
                  pl.BlockSpec((bm, D), lambda i: (i, 0)),
                  pl.BlockSpec((D, ka), lambda i: (0, 0)),
                  pl.BlockSpec((D, D), lambda i: (0, 0)),
                  pl.BlockSpec((1, D), lambda i: (0, 0))],
        out_specs=pl.BlockSpec((n_mol_blk, D), lambda i: (i, 0)),
        out_shape=jax.ShapeDtypeStruct((n // per, D), jnp.float32),
    )(af, a, wa, wb, b)


def _pick(n, pref):
    return pref if n % pref == 0 else n


def kernel(atom_features, f_ini_atoms_bonds, atom_to_incoming_bonds, mapping,
           global_features, molecules_unbatch_key, W_i, W_h, W_o_w, W_o_b):
    n_bonds = f_ini_atoms_bonds.shape[0]
    n_atoms = atom_features.shape[0]
    afdim = atom_features.shape[1]

    cb = _pick(n_bonds, 80)
    ca = _pick(n_atoms, 80)
    bmb = _pick(n_bonds, 2000)
    bma = _pick(n_atoms, 2000)

    idxb_bonds = _pad_chunks(mapping.astype(jnp.int32).reshape(
        n_bonds // cb, cb, 4).transpose(0, 2, 1))
    idxb_atoms = _pad_chunks(atom_to_incoming_bonds.astype(jnp.int32).reshape(
        n_atoms // ca, ca, 4).transpose(0, 2, 1))

    inp, z0 = _mm2(f_ini_atoms_bonds, W_i, W_h, bm=bmb)
    zm = _gsum_sc(z0, idxb_bonds, n_bonds, cb)
    h = _gsum_sc(zm, idxb_bonds, n_bonds, cb, src=inp)
    a = _gsum_sc(h, idxb_atoms, n_atoms, ca)

    wa = W_o_w[:, :afdim]
    wb = W_o_w[:, afdim:]
    mol_sum = _out_stage(atom_features, a, wa, wb,
                         W_o_b.reshape(1, D), bm=bma, per=25)
    mol = mol_sum / molecules_unbatch_key
    return jnp.concatenate([mol, global_features], axis=1)


# speedup vs baseline: 1.8040x; 1.7934x over previous
import functools

import jax
import jax.numpy as jnp
from jax import lax
from jax.experimental import pallas as pl
from jax.experimental.pallas import tpu as pltpu
from jax.experimental.pallas import tpu_sc as plsc

D = 128
LANES = 16
NW = 32


def _gsum_body(*refs, n_chunks, n_iter, chunk, has_src):
    if has_src:
        (table, idxb, src, out, idx_v, rows_v, src_v, acc_v,
         semg0, semg1, semo0, semo1) = refs
    else:
        (table, idxb, out, idx_v, rows_v, acc_v,
         semg0, semg1, semo0, semo1) = refs
        src = src_v = None
    cid = lax.axis_index("c")
    sid = lax.axis_index("s")
    wid = sid * 2 + cid
    semg = (semg0, semg1)
    semo = (semo0, semo1)
    n_acc = 1 if has_src else 2
    nv = D // LANES

    def gather_descs(c, b):
        ds = [pltpu.make_async_copy(table.at[idx_v.at[b, j]],
                                    rows_v.at[b, j], semg[b])
              for j in range(4)]
        if has_src:
            ds.append(pltpu.make_async_copy(
                src.at[pl.ds(c * chunk, chunk)], src_v.at[b], semg[b]))
        return ds

    def out_desc(c, a):
        return pltpu.make_async_copy(
            acc_v.at[a], out.at[pl.ds(c * chunk, chunk)], semo[a])

    def fire(c, b):
        pltpu.sync_copy(idxb.at[c], idx_v.at[b])
        for d in gather_descs(c, b):
            d.start()

    def process(t, c, b):
        a = 0 if has_src else b
        for d in gather_descs(c, b):
            d.wait()

        @pl.when(t >= n_acc)
        def _():
            out_desc(c - n_acc * NW, a).wait()

        def row_body(r, rc):
            for k in range(nv):
                s = pl.ds(k * LANES, LANES)
                vs = [rows_v[b, j, r, s] for j in range(4)]
                v = (vs[0] + vs[1]) + (vs[2] + vs[3])
                if has_src:
                    v = jnp.maximum(v + src_v[b, r, s], 0)
                acc_v[a, r, s] = v
            return rc

        lax.fori_loop(0, chunk, row_body, 0)
        out_desc(c, a).start()

    fire(wid, 0)

    def pair_body(u, carry):
        for b in (0, 1):
            t = u * 2 + b
            c = t * NW + wid
            cn = c + NW

            @pl.when(cn < n_chunks)
            def _():
                fire(cn, 1 - b)

            @pl.when(c < n_chunks)
            def _():
                process(t, c, b)

        return carry

    lax.fori_loop(0, (n_iter + 1) // 2, pair_body, 0)

    t_last = (n_chunks - 1 - wid) // NW
    if has_src:
        out_desc(t_last * NW + wid, 0).wait()
    else:
        for par in (0, 1):
            t_par = t_last - (t_last - par) % 2

            @pl.when(t_par >= 0)
            def _():
                out_desc(t_par * NW + wid, par).wait()


def _gsum_sc(table, idxb, n_out, chunk, src=None):
    n_chunks = n_out // chunk
    n_iter = (n_chunks + NW - 1) // NW
    has_src = src is not None
    mesh = plsc.VectorSubcoreMesh(core_axis_name="c", subcore_axis_name="s",
                                  num_cores=2, num_subcores=16)
    scratch = [
        pltpu.VMEM((2, 4, chunk), jnp.int32),
        pltpu.VMEM((2, 4, chunk, D), jnp.float32),
    ]
    if has_src:
        scratch.append(pltpu.VMEM((2, chunk, D), jnp.float32))
    scratch.append(pltpu.VMEM((1 if has_src else 2, chunk, D), jnp.float32))
    scratch += [pltpu.SemaphoreType.DMA] * 4
    kern = pl.kernel(
        functools.partial(_gsum_body, n_chunks=n_chunks, n_iter=n_iter,
                          chunk=chunk, has_src=has_src),
        out_type=jax.ShapeDtypeStruct((n_out, D), jnp.float32),
        mesh=mesh,
        scratch_types=scratch,
        name=("gsum_add_relu" if has_src else "gsum"),
    )
    return (kern(table, idxb, src) if has_src
            else kern(table, idxb))


def _pad_chunks(idxb):
    n_chunks = idxb.shape[0]
    pw = -(-n_chunks // NW)
    total = NW * (-(-pw // 16)) * 16
    if total > n_chunks:
        idxb = jnp.concatenate(
            [idxb, jnp.zeros((total - n_chunks,) + idxb.shape[1:],
                             jnp.int32)], axis=0)
    return idxb


def _mm2_body(x_ref, wi_ref, wh_ref, inp_ref, z0_ref):
    inp = lax.dot_general(
        x_ref[...], wi_ref[...], (((1,), (1,)), ((), ())),
        preferred_element_type=jnp.float32)
    inp_ref[...] = inp
    z0_ref[...] = lax.dot_general(
        jnp.maximum(inp, 0.0), wh_ref[...], (((1,), (1,)), ((), ())),
        preferred_element_type=jnp.float32)


def _mm2(x, wi, wh, bm):
    n, k = x.shape
    return pl.pallas_call(
        _mm2_body,
        grid=(n // bm,),
        in_specs=[pl.BlockSpec((bm, k), lambda i: (i, 0)),
                  pl.BlockSpec((D, k), lambda i: (0, 0)),
                  pl.BlockSpec((D, D), lambda i: (0, 0))],
        out_specs=[pl.BlockSpec((bm, D), lambda i: (i, 0)),
                   pl.BlockSpec((bm, D), lambda i: (i, 0))],
        out_shape=[jax.ShapeDtypeStruct((n, D), jnp.float32),
                   jax.ShapeDtypeStruct((n, D), jnp.float32)],
    )(x, wi, wh)


def _out_body(af_ref, a_ref, wa_ref, wb_ref, b_ref, o_ref, *, bm, per):
    h = lax.dot_general(af_ref[...], wa_ref[...], (((1,), (1,)), ((), ())),
                        preferred_element_type=jnp.float32)
    h = h + lax.dot_general(a_ref[...], wb_ref[...], (((1,), (1,)), ((), ())),
                            preferred_element_type=jnp.float32)
    h = jnp.maximum(h + b_ref[...], 0.0)
    o_ref[...] = h.reshape(bm // per, per, D).sum(axis=1)


def _out_stage(af, a, wa, wb, b, bm, per):
    n = af.shape[0]
    ka = af.shape[1]
    n_mol_blk = bm // per
    return pl.pallas_call(
        functools.partial(_out_body, bm=bm, per=per),
        grid=(n // bm,),
        in_specs=[pl.BlockSpec((bm, ka), lambda i: (i, 0)),
                  pl.BlockSpec((bm, D), lambda i: (i, 0)),
                  pl.BlockSpec((D, ka), lambda i: (0, 0)),
                  pl.BlockSpec((D, D), lambda i: (0, 0)),
                  pl.BlockSpec((1, D), lambda i: (0, 0))],
        out_specs=pl.BlockSpec((n_mol_blk, D), lambda i: (i, 0)),
        out_shape=jax.ShapeDtypeStruct((n // per, D), jnp.float32),
    )(af, a, wa, wb, b)


def _pick(n, pref):
    return pref if n % pref == 0 else n


def kernel(atom_features, f_ini_atoms_bonds, atom_to_incoming_bonds, mapping,
           global_features, molecules_unbatch_key, W_i, W_h, W_o_w, W_o_b):
    n_bonds = f_ini_atoms_bonds.shape[0]
    n_atoms = atom_features.shape[0]
    afdim = atom_features.shape[1]

    cb = _pick(n_bonds, 80)
    ca = _pick(n_atoms, 80)
    bmb = _pick(n_bonds, 2000)
    bma = _pick(n_atoms, 2000)

    idxb_bonds = _pad_chunks(mapping.astype(jnp.int32).reshape(
        n_bonds // cb, cb, 4).transpose(0, 2, 1))
    idxb_atoms = _pad_chunks(atom_to_incoming_bonds.astype(jnp.int32).reshape(
        n_atoms // ca, ca, 4).transpose(0, 2, 1))

    inp, z0 = _mm2(f_ini_atoms_bonds, W_i, W_h, bm=bmb)
    zm = _gsum_sc(z0, idxb_bonds, n_bonds, cb)
    h = _gsum_sc(zm, idxb_bonds, n_bonds, cb, src=inp)
    a = _gsum_sc(h, idxb_atoms, n_atoms, ca)

    wa = W_o_w[:, :afdim]
    wb = W_o_w[:, afdim:]
    mol_sum = _out_stage(atom_features, a, wa, wb,
                         W_o_b.reshape(1, D), bm=bma, per=25)
    mol = mol_sum / molecules_unbatch_key
    return jnp.concatenate([mol, global_features], axis=1)


# speedup vs baseline: 1.8770x; 1.0405x over previous
import functools

import jax
import jax.numpy as jnp
from jax import lax
from jax.experimental import pallas as pl
from jax.experimental.pallas import tpu as pltpu
from jax.experimental.pallas import tpu_sc as plsc

D = 128
LANES = 16
NW = 32


def _gsum_body(*refs, n_chunks, n_iter, chunk, has_src):
    if has_src:
        (table, idxb, src, out, idx_v, rows_v, src_v, acc_v,
         semg0, semg1, semo0, semo1, semi0, semi1) = refs
    else:
        (table, idxb, out, idx_v, rows_v, acc_v,
         semg0, semg1, semo0, semo1, semi0, semi1) = refs
        src = src_v = None
    cid = lax.axis_index("c")
    sid = lax.axis_index("s")
    wid = sid * 2 + cid
    semg = (semg0, semg1)
    semo = (semo0, semo1)
    semi = (semi0, semi1)
    n_acc = 1 if has_src else 2
    nv = D // LANES

    def gather_descs(c, b):
        ds = [pltpu.make_async_copy(table.at[idx_v.at[b, j]],
                                    rows_v.at[b, j], semg[b])
              for j in range(4)]
        if has_src:
            ds.append(pltpu.make_async_copy(
                src.at[pl.ds(c * chunk, chunk)], src_v.at[b], semg[b]))
        return ds

    def out_desc(c, a):
        return pltpu.make_async_copy(
            acc_v.at[a], out.at[pl.ds(c * chunk, chunk)], semo[a])

    def idx_desc(c, b):
        return pltpu.make_async_copy(idxb.at[c], idx_v.at[b], semi[b])

    def fire(c, b, first=False):
        if first:
            pltpu.sync_copy(idxb.at[c], idx_v.at[b])
        else:
            idx_desc(c, b).wait()
        for d in gather_descs(c, b):
            d.start()

    def process(t, c, b):
        a = 0 if has_src else b
        for d in gather_descs(c, b):
            d.wait()

        @pl.when(c + 2 * NW < n_chunks)
        def _():
            idx_desc(c + 2 * NW, b).start()

        @pl.when(t >= n_acc)
        def _():
            out_desc(c - n_acc * NW, a).wait()

        def row_body(r, rc):
            for k in range(nv):
                s = pl.ds(k * LANES, LANES)
                vs = [rows_v[b, j, r, s] for j in range(4)]
                v = (vs[0] + vs[1]) + (vs[2] + vs[3])
                if has_src:
                    v = jnp.maximum(v + src_v[b, r, s], 0)
                acc_v[a, r, s] = v
            return rc

        lax.fori_loop(0, chunk, row_body, 0)
        out_desc(c, a).start()

    fire(wid, 0, first=True)

    @pl.when(wid + NW < n_chunks)
    def _():
        idx_desc(wid + NW, 1).start()

    def pair_body(u, carry):
        for b in (0, 1):
            t = u * 2 + b
            c = t * NW + wid
            cn = c + NW

            @pl.when(cn < n_chunks)
            def _():
                fire(cn, 1 - b)

            @pl.when(c < n_chunks)
            def _():
                process(t, c, b)

        return carry

    lax.fori_loop(0, (n_iter + 1) // 2, pair_body, 0)

    t_last = (n_chunks - 1 - wid) // NW
    if has_src:
        out_desc(t_last * NW + wid, 0).wait()
    else:
        for par in (0, 1):
            t_par = t_last - (t_last - par) % 2

            @pl.when(t_par >= 0)
            def _():
                out_desc(t_par * NW + wid, par).wait()


def _gsum_sc(table, idxb, n_out, chunk, src=None):
    n_chunks = n_out // chunk
    n_iter = (n_chunks + NW - 1) // NW
    has_src = src is not None
    mesh = plsc.VectorSubcoreMesh(core_axis_name="c", subcore_axis_name="s",
                                  num_cores=2, num_subcores=16)
    scratch = [
        pltpu.VMEM((2, 4, chunk), jnp.int32),
        pltpu.VMEM((2, 4, chunk, D), jnp.float32),
    ]
    if has_src:
        scratch.append(pltpu.VMEM((2, chunk, D), jnp.float32))
    scratch.append(pltpu.VMEM((1 if has_src else 2, chunk, D), jnp.float32))
    scratch += [pltpu.SemaphoreType.DMA] * 6
    kern = pl.kernel(
        functools.partial(_gsum_body, n_chunks=n_chunks, n_iter=n_iter,
                          chunk=chunk, has_src=has_src),
        out_type=jax.ShapeDtypeStruct((n_out, D), jnp.float32),
        mesh=mesh,
        scratch_types=scratch,
        name=("gsum_add_relu" if has_src else "gsum"),
    )
    return (kern(table, idxb, src) if has_src
            else kern(table, idxb))


def _pad_chunks(idxb):
    n_chunks = idxb.shape[0]
    pw = -(-n_chunks // NW)
    total = NW * (-(-pw // 16)) * 16
    if total > n_chunks:
        idxb = jnp.concatenate(
            [idxb, jnp.zeros((total - n_chunks,) + idxb.shape[1:],
                             jnp.int32)], axis=0)
    return idxb


def _mm2_body(x_ref, wi_ref, wh_ref, inp_ref, z0_ref):
    inp = lax.dot_general(
        x_ref[...], wi_ref[...], (((1,), (1,)), ((), ())),
        preferred_element_type=jnp.float32)
    inp_ref[...] = inp
    z0_ref[...] = lax.dot_general(
        jnp.maximum(inp, 0.0), wh_ref[...], (((1,), (1,)), ((), ())),
        preferred_element_type=jnp.float32)


def _mm2(x, wi, wh, bm):
    n, k = x.shape
    return pl.pallas_call(
        _mm2_body,
        grid=(n // bm,),
        in_specs=[pl.BlockSpec((bm, k), lambda i: (i, 0)),
                  pl.BlockSpec((D, k), lambda i: (0, 0)),
                  pl.BlockSpec((D, D), lambda i: (0, 0))],
        out_specs=[pl.BlockSpec((bm, D), lambda i: (i, 0)),
                   pl.BlockSpec((bm, D), lambda i: (i, 0))],
        out_shape=[jax.ShapeDtypeStruct((n, D), jnp.float32),
                   jax.ShapeDtypeStruct((n, D), jnp.float32)],
    )(x, wi, wh)


def _out_body(af_ref, a_ref, wa_ref, wb_ref, b_ref, o_ref, *, bm, per):
    h = lax.dot_general(af_ref[...], wa_ref[...], (((1,), (1,)), ((), ())),
                        preferred_element_type=jnp.float32)
    h = h + lax.dot_general(a_ref[...], wb_ref[...], (((1,), (1,)), ((), ())),
                            preferred_element_type=jnp.float32)
    h = jnp.maximum(h + b_ref[...], 0.0)
    o_ref[...] = h.reshape(bm // per, per, D).sum(axis=1)


def _out_stage(af, a, wa, wb, b, bm, per):
    n = af.shape[0]
    ka = af.shape[1]
    n_mol_blk = bm // per
    return pl.pallas_call(
        functools.partial(_out_body, bm=bm, per=per),
        grid=(n // bm,),
        in_specs=[pl.BlockSpec((bm, ka), lambda i: (i, 0)),
                  pl.BlockSpec((bm, D), lambda i: (i, 0)),
                  pl.BlockSpec((D, ka), lambda i: (0, 0)),
                  pl.BlockSpec((D, D), lambda i: (0, 0)),
                  pl.BlockSpec((1, D), lambda i: (0, 0))],
        out_specs=pl.BlockSpec((n_mol_blk, D), lambda i: (i, 0)),
        out_shape=jax.ShapeDtypeStruct((n // per, D), jnp.float32),
    )(af, a, wa, wb, b)


def _pick(n, pref):
    return pref if n % pref == 0 else n


def kernel(atom_features, f_ini_atoms_bonds, atom_to_incoming_bonds, mapping,
           global_features, molecules_unbatch_key, W_i, W_h, W_o_w, W_o_b):
    n_bonds = f_ini_atoms_bonds.shape[0]
    n_atoms = atom_features.shape[0]
    afdim = atom_features.shape[1]

    cb = _pick(n_bonds, 80)
    ca = _pick(n_atoms, 80)
    bmb = _pick(n_bonds, 2000)
    bma = _pick(n_atoms, 2000)

    idxb_bonds = _pad_chunks(mapping.astype(jnp.int32).reshape(
        n_bonds // cb, cb, 4).transpose(0, 2, 1))
    idxb_atoms = _pad_chunks(atom_to_incoming_bonds.astype(jnp.int32).reshape(
        n_atoms // ca, ca, 4).transpose(0, 2, 1))

    inp, z0 = _mm2(f_ini_atoms_bonds, W_i, W_h, bm=bmb)
    zm = _gsum_sc(z0, idxb_bonds, n_bonds, cb)
    h = _gsum_sc(zm, idxb_bonds, n_bonds, cb, src=inp)
    a = _gsum_sc(h, idxb_atoms, n_atoms, ca)

    wa = W_o_w[:, :afdim]
    wb = W_o_w[:, afdim:]
    mol_sum = _out_stage(atom_features, a, wa, wb,
                         W_o_b.reshape(1, D), bm=bma, per=25)
    mol = mol_sum / molecules_unbatch_key
    return jnp.concatenate([mol, global_features], axis=1)


# speedup vs baseline: 1.9531x; 1.0405x over previous
import functools

import jax
import jax.numpy as jnp
from jax import lax
from jax.experimental import pallas as pl
from jax.experimental.pallas import tpu as pltpu
from jax.experimental.pallas import tpu_sc as plsc

D = 128
LANES = 16
NW = 32


def _gsum_body(*refs, n_chunks, n_iter, chunk, has_src):
    if has_src:
        (table, idxb, src, out, idx_v, rows_v, src_v, acc_v,
         semg0, semg1, semo0, semo1, semi0, semi1) = refs
    else:
        (table, idxb, out, idx_v, rows_v, acc_v,
         semg0, semg1, semo0, semo1, semi0, semi1) = refs
        src = src_v = None
    cid = lax.axis_index("c")
    sid = lax.axis_index("s")
    wid = sid * 2 + cid
    semg = (semg0, semg1)
    semo = (semo0, semo1)
    semi = (semi0, semi1)
    n_acc = 1 if has_src else 2
    nv = D // LANES

    def gather_descs(c, b):
        ds = [pltpu.make_async_copy(table.at[idx_v.at[b, j]],
                                    rows_v.at[b, j], semg[b])
              for j in range(4)]
        if has_src:
            ds.append(pltpu.make_async_copy(
                src.at[pl.ds(c * chunk, chunk)], src_v.at[b], semg[b]))
        return ds

    def out_desc(c, a):
        return pltpu.make_async_copy(
            acc_v.at[a], out.at[pl.ds(c * chunk, chunk)], semo[a])

    def idx_desc(c, b):
        return pltpu.make_async_copy(idxb.at[c], idx_v.at[b], semi[b])

    def fire(c, b, first=False):
        if first:
            pltpu.sync_copy(idxb.at[c], idx_v.at[b])
        else:
            idx_desc(c, b).wait()
        for d in gather_descs(c, b):
            d.start()

    def process(t, c, b):
        a = 0 if has_src else b
        for d in gather_descs(c, b):
            d.wait()

        @pl.when(c + 2 * NW < n_chunks)
        def _():
            idx_desc(c + 2 * NW, b).start()

        @pl.when(t >= n_acc)
        def _():
            out_desc(c - n_acc * NW, a).wait()

        def row_body(r, rc):
            for k in range(nv):
                s = pl.ds(k * LANES, LANES)
                vs = [rows_v[b, j, r, s] for j in range(4)]
                v = (vs[0] + vs[1]) + (vs[2] + vs[3])
                if has_src:
                    v = jnp.maximum(v + src_v[b, r, s], 0)
                acc_v[a, r, s] = v
            return rc

        lax.fori_loop(0, chunk, row_body, 0)
        out_desc(c, a).start()

    fire(wid, 0, first=True)

    @pl.when(wid + NW < n_chunks)
    def _():
        idx_desc(wid + NW, 1).start()

    def pair_body(u, carry):
        for b in (0, 1):
            t = u * 2 + b
            c = t * NW + wid
            cn = c + NW

            @pl.when(cn < n_chunks)
            def _():
                fire(cn, 1 - b)

            @pl.when(c < n_chunks)
            def _():
                process(t, c, b)

        return carry

    lax.fori_loop(0, (n_iter + 1) // 2, pair_body, 0)

    t_last = (n_chunks - 1 - wid) // NW
    if has_src:
        out_desc(t_last * NW + wid, 0).wait()
    else:
        for par in (0, 1):
            t_par = t_last - (t_last - par) % 2

            @pl.when(t_par >= 0)
            def _():
                out_desc(t_par * NW + wid, par).wait()


def _gsum_sc(table, idxb, n_out, chunk, src=None):
    n_chunks = n_out // chunk
    n_iter = (n_chunks + NW - 1) // NW
    has_src = src is not None
    mesh = plsc.VectorSubcoreMesh(core_axis_name="c", subcore_axis_name="s",
                                  num_cores=2, num_subcores=16)
    scratch = [
        pltpu.VMEM((2, 4, chunk), jnp.int32),
        pltpu.VMEM((2, 4, chunk, D), jnp.float32),
    ]
    if has_src:
        scratch.append(pltpu.VMEM((2, chunk, D), jnp.float32))
    scratch.append(pltpu.VMEM((1 if has_src else 2, chunk, D), jnp.float32))
    scratch += [pltpu.SemaphoreType.DMA] * 6
    kern = pl.kernel(
        functools.partial(_gsum_body, n_chunks=n_chunks, n_iter=n_iter,
                          chunk=chunk, has_src=has_src),
        out_type=jax.ShapeDtypeStruct((n_out, D), jnp.float32),
        mesh=mesh,
        scratch_types=scratch,
        name=("gsum_add_relu" if has_src else "gsum"),
    )
    return (kern(table, idxb, src) if has_src
            else kern(table, idxb))


def _pad_chunks(idxb):
    n_chunks = idxb.shape[0]
    pw = -(-n_chunks // NW)
    total = NW * (-(-pw // 16)) * 16
    if total > n_chunks:
        idxb = jnp.concatenate(
            [idxb, jnp.zeros((total - n_chunks,) + idxb.shape[1:],
                             jnp.int32)], axis=0)
    return idxb


def _mm2_body(x_ref, wi_ref, wh_ref, inp_ref, z0_ref):
    inp = lax.dot_general(
        x_ref[...], wi_ref[...], (((1,), (1,)), ((), ())),
        preferred_element_type=jnp.float32)
    inp_ref[...] = inp
    z0_ref[...] = lax.dot_general(
        jnp.maximum(inp, 0.0), wh_ref[...], (((1,), (1,)), ((), ())),
        preferred_element_type=jnp.float32)


def _mm2(x, wi, wh, bm):
    n, k = x.shape
    return pl.pallas_call(
        _mm2_body,
        grid=(n // bm,),
        in_specs=[pl.BlockSpec((bm, k), lambda i: (i, 0)),
                  pl.BlockSpec((D, k), lambda i: (0, 0)),
                  pl.BlockSpec((D, D), lambda i: (0, 0))],
        out_specs=[pl.BlockSpec((bm, D), lambda i: (i, 0)),
                   pl.BlockSpec((bm, D), lambda i: (i, 0))],
        out_shape=[jax.ShapeDtypeStruct((n, D), jnp.float32),
                   jax.ShapeDtypeStruct((n, D), jnp.float32)],
    )(x, wi, wh)


def _out_body(af_ref, a_ref, wa_ref, wb_ref, b_ref, o_ref, *, bm, per):
    h = lax.dot_general(af_ref[...], wa_ref[...], (((1,), (1,)), ((), ())),
                        preferred_element_type=jnp.float32)
    h = h + lax.dot_general(a_ref[...], wb_ref[...], (((1,), (1,)), ((), ())),
                            preferred_element_type=jnp.float32)
    h = jnp.maximum(h + b_ref[...], 0.0)
    o_ref[...] = h.reshape(bm // per, per, D).sum(axis=1)


def _out_stage(af, a, wa, wb, b, bm, per):
    n = af.shape[0]
    ka = af.shape[1]
    n_mol_blk = bm // per
    return pl.pallas_call(
        functools.partial(_out_body, bm=bm, per=per),
        grid=(n // bm,),
        in_specs=[pl.BlockSpec((bm, ka), lambda i: (i, 0)),
                  pl.BlockSpec((bm, D), lambda i: (i, 0)),
                  pl.BlockSpec((D, ka), lambda i: (0, 0)),
                  pl.BlockSpec((D, D), lambda i: (0, 0)),
                  pl.BlockSpec((1, D), lambda i: (0, 0))],
        out_specs=pl.BlockSpec((n_mol_blk, D), lambda i: (i, 0)),
        out_shape=jax.ShapeDtypeStruct((n // per, D), jnp.float32),
    )(af, a, wa, wb, b)


def _pick(n, pref):
    return pref if n % pref == 0 else n


def kernel(atom_features, f_ini_atoms_bonds, atom_to_incoming_bonds, mapping,
           global_features, molecules_unbatch_key, W_i, W_h, W_o_w, W_o_b):
    n_bonds = f_ini_atoms_bonds.shape[0]
    n_atoms = atom_features.shape[0]
    afdim = atom_features.shape[1]

    cb = _pick(n_bonds, 80)
    ca = _pick(n_atoms, 80)
    bmb = _pick(n_bonds, 5000)
    bma = _pick(n_atoms, 2000)

    idxb_bonds = _pad_chunks(mapping.astype(jnp.int32).reshape(
        n_bonds // cb, cb, 4).transpose(0, 2, 1))
    idxb_atoms = _pad_chunks(atom_to_incoming_bonds.astype(jnp.int32).reshape(
        n_atoms // ca, ca, 4).transpose(0, 2, 1))

    inp, z0 = _mm2(f_ini_atoms_bonds, W_i, W_h, bm=bmb)
    zm = _gsum_sc(z0, idxb_bonds, n_bonds, cb)
    h = _gsum_sc(zm, idxb_bonds, n_bonds, cb, src=inp)
    a = _gsum_sc(h, idxb_atoms, n_atoms, ca)

    wa = W_o_w[:, :afdim]
    wb = W_o_w[:, afdim:]
    mol_sum = _out_stage(atom_features, a, wa, wb,
                         W_o_b.reshape(1, D), bm=bma, per=25)
    mol = mol_sum / molecules_unbatch_key
    return jnp.concatenate([mol, global_features], axis=1)


# speedup vs baseline: 1.9805x; 1.0140x over previous
import functools

import jax
import jax.numpy as jnp
from jax import lax
from jax.experimental import pallas as pl
from jax.experimental.pallas import tpu as pltpu
from jax.experimental.pallas import tpu_sc as plsc

D = 128
LANES = 16
NW = 32


def _gsum_body(*refs, n_chunks, n_iter, chunk, has_src):
    if has_src:
        (table, idxb, src, out, idx_v, rows_v, src_v, acc_v,
         semg0, semg1, semo0, semo1, semi0, semi1) = refs
    else:
        (table, idxb, out, idx_v, rows_v, acc_v,
         semg0, semg1, semo0, semo1, semi0, semi1) = refs
        src = src_v = None
    cid = lax.axis_index("c")
    sid = lax.axis_index("s")
    wid = sid * 2 + cid
    semg = (semg0, semg1)
    semo = (semo0, semo1)
    semi = (semi0, semi1)
    n_acc = 1 if has_src else 2
    nv = D // LANES

    def gather_descs(c, b):
        ds = [pltpu.make_async_copy(table.at[idx_v.at[b, j]],
                                    rows_v.at[b, j], semg[b])
              for j in range(4)]
        if has_src:
            ds.append(pltpu.make_async_copy(
                src.at[pl.ds(c * chunk, chunk)], src_v.at[b], semg[b]))
        return ds

    def out_desc(c, a):
        return pltpu.make_async_copy(
            acc_v.at[a], out.at[pl.ds(c * chunk, chunk)], semo[a])

    def idx_desc(c, b):
        return pltpu.make_async_copy(idxb.at[c], idx_v.at[b], semi[b])

    def fire(c, b, first=False):
        if first:
            pltpu.sync_copy(idxb.at[c], idx_v.at[b])
        else:
            idx_desc(c, b).wait()
        for d in gather_descs(c, b):
            d.start()

    def process(t, c, b):
        a = 0 if has_src else b
        for d in gather_descs(c, b):
            d.wait()

        @pl.when(c + 2 * NW < n_chunks)
        def _():
            idx_desc(c + 2 * NW, b).start()

        @pl.when(t >= n_acc)
        def _():
            out_desc(c - n_acc * NW, a).wait()

        def row_body(r, rc):
            for k in range(nv):
                s = pl.ds(k * LANES, LANES)
                vs = [rows_v[b, j, r, s] for j in range(4)]
                v = (vs[0] + vs[1]) + (vs[2] + vs[3])
                if has_src:
                    v = jnp.maximum(v + src_v[b, r, s], 0)
                acc_v[a, r, s] = v
            return rc

        lax.fori_loop(0, chunk, row_body, 0)
        out_desc(c, a).start()

    fire(wid, 0, first=True)

    @pl.when(wid + NW < n_chunks)
    def _():
        idx_desc(wid + NW, 1).start()

    def pair_body(u, carry):
        for b in (0, 1):
            t = u * 2 + b
            c = t * NW + wid
            cn = c + NW

            @pl.when(cn < n_chunks)
            def _():
                fire(cn, 1 - b)

            @pl.when(c < n_chunks)
            def _():
                process(t, c, b)

        return carry

    lax.fori_loop(0, (n_iter + 1) // 2, pair_body, 0)

    t_last = (n_chunks - 1 - wid) // NW
    if has_src:
        out_desc(t_last * NW + wid, 0).wait()
    else:
        for par in (0, 1):
            t_par = t_last - (t_last - par) % 2

            @pl.when(t_par >= 0)
            def _():
                out_desc(t_par * NW + wid, par).wait()


def _gsum_sc(table, idxb, n_out, chunk, src=None):
    n_chunks = n_out // chunk
    n_iter = (n_chunks + NW - 1) // NW
    has_src = src is not None
    mesh = plsc.VectorSubcoreMesh(core_axis_name="c", subcore_axis_name="s",
                                  num_cores=2, num_subcores=16)
    scratch = [
        pltpu.VMEM((2, 4, chunk), jnp.int32),
        pltpu.VMEM((2, 4, chunk, D), jnp.float32),
    ]
    if has_src:
        scratch.append(pltpu.VMEM((2, chunk, D), jnp.float32))
    scratch.append(pltpu.VMEM((1 if has_src else 2, chunk, D), jnp.float32))
    scratch += [pltpu.SemaphoreType.DMA] * 6
    kern = pl.kernel(
        functools.partial(_gsum_body, n_chunks=n_chunks, n_iter=n_iter,
                          chunk=chunk, has_src=has_src),
        out_type=jax.ShapeDtypeStruct((n_out, D), jnp.float32),
        mesh=mesh,
        scratch_types=scratch,
        name=("gsum_add_relu" if has_src else "gsum"),
    )
    return (kern(table, idxb, src) if has_src
            else kern(table, idxb))


def _pad_chunks(idxb):
    n_chunks = idxb.shape[0]
    pw = -(-n_chunks // NW)
    total = NW * (-(-pw // 16)) * 16
    if total > n_chunks:
        idxb = jnp.concatenate(
            [idxb, jnp.zeros((total - n_chunks,) + idxb.shape[1:],
                             jnp.int32)], axis=0)
    return idxb


def _mm2_body(x_ref, wi_ref, wh_ref, inp_ref, z0_ref):
    inp = lax.dot_general(
        x_ref[...], wi_ref[...], (((1,), (1,)), ((), ())),
        preferred_element_type=jnp.float32)
    inp_ref[...] = inp
    z0_ref[...] = lax.dot_general(
        jnp.maximum(inp, 0.0), wh_ref[...], (((1,), (1,)), ((), ())),
        preferred_element_type=jnp.float32)


def _mm2(x, wi, wh, bm):
    n, k = x.shape
    return pl.pallas_call(
        _mm2_body,
        grid=(n // bm,),
        in_specs=[pl.BlockSpec((bm, k), lambda i: (i, 0)),
                  pl.BlockSpec((D, k), lambda i: (0, 0)),
                  pl.BlockSpec((D, D), lambda i: (0, 0))],
        out_specs=[pl.BlockSpec((bm, D), lambda i: (i, 0)),
                   pl.BlockSpec((bm, D), lambda i: (i, 0))],
        out_shape=[jax.ShapeDtypeStruct((n, D), jnp.float32),
                   jax.ShapeDtypeStruct((n, D), jnp.float32)],
    )(x, wi, wh)


def _out_body(af_ref, a_ref, wa_ref, wb_ref, b_ref, o_ref, *, bm, per):
    h = lax.dot_general(af_ref[...], wa_ref[...], (((1,), (1,)), ((), ())),
                        preferred_element_type=jnp.float32)
    h = h + lax.dot_general(a_ref[...], wb_ref[...], (((1,), (1,)), ((), ())),
                            preferred_element_type=jnp.float32)
    h = jnp.maximum(h + b_ref[...], 0.0)
    o_ref[...] = h.reshape(bm // per, per, D).sum(axis=1)


def _out_stage(af, a, wa, wb, b, bm, per):
    n = af.shape[0]
    ka = af.shape[1]
    n_mol_blk = bm // per
    return pl.pallas_call(
        functools.partial(_out_body, bm=bm, per=per),
        grid=(n // bm,),
        in_specs=[pl.BlockSpec((bm, ka), lambda i: (i, 0)),
                  pl.BlockSpec((bm, D), lambda i: (i, 0)),
                  pl.BlockSpec((D, ka), lambda i: (0, 0)),
                  pl.BlockSpec((D, D), lambda i: (0, 0)),
                  pl.BlockSpec((1, D), lambda i: (0, 0))],
        out_specs=pl.BlockSpec((n_mol_blk, D), lambda i: (i, 0)),
        out_shape=jax.ShapeDtypeStruct((n // per, D), jnp.float32),
    )(af, a, wa, wb, b)


def _pick(n, pref):
    return pref if n % pref == 0 else n


def kernel(atom_features, f_ini_atoms_bonds, atom_to_incoming_bonds, mapping,
           global_features, molecules_unbatch_key, W_i, W_h, W_o_w, W_o_b):
    n_bonds = f_ini_atoms_bonds.shape[0]
    n_atoms = atom_features.shape[0]
    afdim = atom_features.shape[1]

    cb = _pick(n_bonds, 80)
    ca = _pick(n_atoms, 80)
    bmb = _pick(n_bonds, 10000)
    bma = _pick(n_atoms, 5000)

    idxb_bonds = _pad_chunks(mapping.astype(jnp.int32).reshape(
        n_bonds // cb, cb, 4).transpose(0, 2, 1))
    idxb_atoms = _pad_chunks(atom_to_incoming_bonds.astype(jnp.int32).reshape(
        n_atoms // ca, ca, 4).transpose(0, 2, 1))

    inp, z0 = _mm2(f_ini_atoms_bonds, W_i, W_h, bm=bmb)
    zm = _gsum_sc(z0, idxb_bonds, n_bonds, cb)
    h = _gsum_sc(zm, idxb_bonds, n_bonds, cb, src=inp)
    a = _gsum_sc(h, idxb_atoms, n_atoms, ca)

    wa = W_o_w[:, :afdim]
    wb = W_o_w[:, afdim:]
    mol_sum = _out_stage(atom_features, a, wa, wb,
                         W_o_b.reshape(1, D), bm=bma, per=25)
    mol = mol_sum / molecules_unbatch_key
    return jnp.concatenate([mol, global_features], axis=1)


# speedup vs baseline: 1.9955x; 1.0076x over previous
import functools

import jax
import jax.numpy as jnp
from jax import lax
from jax.experimental import pallas as pl
from jax.experimental.pallas import tpu as pltpu
from jax.experimental.pallas import tpu_sc as plsc

D = 128
LANES = 16
NW = 32


def _gsum_body(*refs, n_chunks, n_iter, chunk, has_src):
    if has_src:
        (table, idxb, src, out, idx_v, rows_v, src_v, acc_v,
         semg0, semg1, semo0, semo1, semi0, semi1) = refs
    else:
        (table, idxb, out, idx_v, rows_v, acc_v,
         semg0, semg1, semo0, semo1, semi0, semi1) = refs
        src = src_v = None
    cid = lax.axis_index("c")
    sid = lax.axis_index("s")
    wid = sid * 2 + cid
    semg = (semg0, semg1)
    semo = (semo0, semo1)
    semi = (semi0, semi1)
    n_acc = 1 if has_src else 2
    nv = D // LANES

    def gather_descs(c, b):
        ds = [pltpu.make_async_copy(table.at[idx_v.at[b, j]],
                                    rows_v.at[b, j], semg[b])
              for j in range(4)]
        if has_src:
            ds.append(pltpu.make_async_copy(
                src.at[pl.ds(c * chunk, chunk)], src_v.at[b], semg[b]))
        return ds

    def out_desc(c, a):
        return pltpu.make_async_copy(
            acc_v.at[a], out.at[pl.ds(c * chunk, chunk)], semo[a])

    def idx_desc(c, b):
        return pltpu.make_async_copy(idxb.at[c], idx_v.at[b], semi[b])

    def fire(c, b, first=False):
        if first:
            pltpu.sync_copy(idxb.at[c], idx_v.at[b])
        else:
            idx_desc(c, b).wait()
        for d in gather_descs(c, b):
            d.start()

    def process(t, c, b):
        a = 0 if has_src else b
        for d in gather_descs(c, b):
            d.wait()

        @pl.when(c + 2 * NW < n_chunks)
        def _():
            idx_desc(c + 2 * NW, b).start()

        @pl.when(t >= n_acc)
        def _():
            out_desc(c - n_acc * NW, a).wait()

        def row_body(r, rc):
            for k in range(nv):
                s = pl.ds(k * LANES, LANES)
                vs = [rows_v[b, j, r, s] for j in range(4)]
                v = (vs[0] + vs[1]) + (vs[2] + vs[3])
                if has_src:
                    v = jnp.maximum(v + src_v[b, r, s], 0)
                acc_v[a, r, s] = v
            return rc

        lax.fori_loop(0, chunk, row_body, 0)
        out_desc(c, a).start()

    fire(wid, 0, first=True)

    @pl.when(wid + NW < n_chunks)
    def _():
        idx_desc(wid + NW, 1).start()

    def pair_body(u, carry):
        for b in (0, 1):
            t = u * 2 + b
            c = t * NW + wid
            cn = c + NW

            @pl.when(cn < n_chunks)
            def _():
                fire(cn, 1 - b)

            @pl.when(c < n_chunks)
            def _():
                process(t, c, b)

        return carry

    lax.fori_loop(0, (n_iter + 1) // 2, pair_body, 0)

    t_last = (n_chunks - 1 - wid) // NW
    if has_src:
        out_desc(t_last * NW + wid, 0).wait()
    else:
        for par in (0, 1):
            t_par = t_last - (t_last - par) % 2

            @pl.when(t_par >= 0)
            def _():
                out_desc(t_par * NW + wid, par).wait()


def _gsum_sc(table, idxb, n_out, chunk, src=None):
    n_chunks = n_out // chunk
    n_iter = (n_chunks + NW - 1) // NW
    has_src = src is not None
    mesh = plsc.VectorSubcoreMesh(core_axis_name="c", subcore_axis_name="s",
                                  num_cores=2, num_subcores=16)
    scratch = [
        pltpu.VMEM((2, 4, chunk), jnp.int32),
        pltpu.VMEM((2, 4, chunk, D), jnp.float32),
    ]
    if has_src:
        scratch.append(pltpu.VMEM((2, chunk, D), jnp.float32))
    scratch.append(pltpu.VMEM((1 if has_src else 2, chunk, D), jnp.float32))
    scratch += [pltpu.SemaphoreType.DMA] * 6
    kern = pl.kernel(
        functools.partial(_gsum_body, n_chunks=n_chunks, n_iter=n_iter,
                          chunk=chunk, has_src=has_src),
        out_type=jax.ShapeDtypeStruct((n_out, D), jnp.float32),
        mesh=mesh,
        scratch_types=scratch,
        name=("gsum_add_relu" if has_src else "gsum"),
    )
    return (kern(table, idxb, src) if has_src
            else kern(table, idxb))


def _mm2_body(x_ref, wi_ref, wh_ref, inp_ref, z0_ref):
    inp = lax.dot_general(
        x_ref[...], wi_ref[...], (((1,), (1,)), ((), ())),
        preferred_element_type=jnp.float32)
    inp_ref[...] = inp
    z0_ref[...] = lax.dot_general(
        jnp.maximum(inp, 0.0), wh_ref[...], (((1,), (1,)), ((), ())),
        preferred_element_type=jnp.float32)


def _mm2(x, wi, wh, bm):
    n, k = x.shape
    return pl.pallas_call(
        _mm2_body,
        grid=(n // bm,),
        in_specs=[pl.BlockSpec((bm, k), lambda i: (i, 0)),
                  pl.BlockSpec((D, k), lambda i: (0, 0)),
                  pl.BlockSpec((D, D), lambda i: (0, 0))],
        out_specs=[pl.BlockSpec((bm, D), lambda i: (i, 0)),
                   pl.BlockSpec((bm, D), lambda i: (i, 0))],
        out_shape=[jax.ShapeDtypeStruct((n, D), jnp.float32),
                   jax.ShapeDtypeStruct((n, D), jnp.float32)],
    )(x, wi, wh)


def _out_body(af_ref, a_ref, wa_ref, wb_ref, b_ref, o_ref, *, bm, per):
    h = lax.dot_general(af_ref[...], wa_ref[...], (((1,), (1,)), ((), ())),
                        preferred_element_type=jnp.float32)
    h = h + lax.dot_general(a_ref[...], wb_ref[...], (((1,), (1,)), ((), ())),
                            preferred_element_type=jnp.float32)
    h = jnp.maximum(h + b_ref[...], 0.0)
    o_ref[...] = h.reshape(bm // per, per, D).sum(axis=1)


def _out_stage(af, a, wa, wb, b, bm, per):
    n = af.shape[0]
    ka = af.shape[1]
    n_mol_blk = bm // per
    return pl.pallas_call(
        functools.partial(_out_body, bm=bm, per=per),
        grid=(n // bm,),
        in_specs=[pl.BlockSpec((bm, ka), lambda i: (i, 0)),
                  pl.BlockSpec((bm, D), lambda i: (i, 0)),
                  pl.BlockSpec((D, ka), lambda i: (0, 0)),
                  pl.BlockSpec((D, D), lambda i: (0, 0)),
                  pl.BlockSpec((1, D), lambda i: (0, 0))],
        out_specs=pl.BlockSpec((n_mol_blk, D), lambda i: (i, 0)),
        out_shape=jax.ShapeDtypeStruct((n // per, D), jnp.float32),
    )(af, a, wa, wb, b)


def _pick(n, pref):
    return pref if n % pref == 0 else n


def kernel(atom_features, f_ini_atoms_bonds, atom_to_incoming_bonds, mapping,
           global_features, molecules_unbatch_key, W_i, W_h, W_o_w, W_o_b):
    n_bonds = f_ini_atoms_bonds.shape[0]
    n_atoms = atom_features.shape[0]
    afdim = atom_features.shape[1]

    cb = _pick(n_bonds, 80)
    ca = _pick(n_atoms, 80)
    bmb = _pick(n_bonds, 10000)
    bma = _pick(n_atoms, 5000)

    idxb_bonds = mapping.astype(jnp.int32).reshape(
        n_bonds // cb, cb, 4).transpose(0, 2, 1)
    idxb_atoms = atom_to_incoming_bonds.astype(jnp.int32).reshape(
        n_atoms // ca, ca, 4).transpose(0, 2, 1)

    inp, z0 = _mm2(f_ini_atoms_bonds, W_i, W_h, bm=bmb)
    zm = _gsum_sc(z0, idxb_bonds, n_bonds, cb)
    h = _gsum_sc(zm, idxb_bonds, n_bonds, cb, src=inp)
    a = _gsum_sc(h, idxb_atoms, n_atoms, ca)

    wa = W_o_w[:, :afdim]
    wb = W_o_w[:, afdim:]
    mol_sum = _out_stage(atom_features, a, wa, wb,
                         W_o_b.reshape(1, D), bm=bma, per=25)
    mol = mol_sum / molecules_unbatch_key
    return jnp.concatenate([mol, global_features], axis=1)
